# baseline (speedup 1.0000x reference)
import functools

import jax
import jax.numpy as jnp
from jax import lax
from jax.experimental import pallas as pl
from jax.experimental.pallas import tpu as pltpu
from jax.experimental.pallas import tpu_sc as plsc

BATCH = 16384
D = 64
L = 16
NC = 2
NS = 16
NW = NC * NS
CHUNK = BATCH // NW
NIDX = 128
NCH = CHUNK // NIDX

_mesh = plsc.VectorSubcoreMesh(core_axis_name="c", subcore_axis_name="s")


@functools.partial(
    pl.kernel,
    mesh=_mesh,
    compiler_params=pltpu.CompilerParams(
        needs_layout_passes=False, use_tc_tiling_on_sc=False),
    out_type=jax.ShapeDtypeStruct((BATCH,), jnp.float32),
    scratch_types=[
        pltpu.VMEM((NCH, NIDX), jnp.int32),
        pltpu.VMEM((NCH, NIDX), jnp.int32),
        pltpu.VMEM((CHUNK, D), jnp.float32),
        pltpu.VMEM((CHUNK, D), jnp.float32),
        pltpu.VMEM((CHUNK,), jnp.float32),
        pltpu.SemaphoreType.DMA,
    ],
)
def _rec_sc(uidx_hbm, iidx_hbm, utab_hbm, itab_hbm, out_hbm,
            uidx_v, iidx_v, urows_v, irows_v, res_v, sem):
    wid = lax.axis_index("s") * NC + lax.axis_index("c")
    base = wid * CHUNK

    for j in range(NCH):
        pltpu.sync_copy(uidx_hbm.at[pl.ds(base + j * NIDX, NIDX)], uidx_v.at[j])
        pltpu.sync_copy(iidx_hbm.at[pl.ds(base + j * NIDX, NIDX)], iidx_v.at[j])

    copies = []
    for j in range(NCH):
        copies.append(pltpu.async_copy(
            utab_hbm.at[uidx_v.at[j]], urows_v.at[pl.ds(j * NIDX, NIDX)], sem))
        copies.append(pltpu.async_copy(
            itab_hbm.at[iidx_v.at[j]], irows_v.at[pl.ds(j * NIDX, NIDX)], sem))
    for c in copies:
        c.wait()

    zero = jnp.zeros((L,), jnp.float32)
    iota16 = lax.iota(jnp.int32, L)

    def grp_body(g, _):
        out16 = zero
        for jj in range(L):
            r = g * L + jj
            acc = zero
            for c in range(D // L):
                u = jnp.maximum(urows_v[r, pl.ds(c * L, L)], 0.0)
                it = jnp.maximum(irows_v[r, pl.ds(c * L, L)], 0.0)
                acc = acc + u * it
            out16 = jnp.where(iota16 == jj, jnp.sum(acc), out16)
        res_v[pl.ds(g * L, L)] = 4.0 / (1.0 + jnp.exp(-out16)) + 1.0
        return 0

    lax.fori_loop(0, CHUNK // L, grp_body, 0)

    pltpu.sync_copy(res_v, out_hbm.at[pl.ds(base, CHUNK)])


def kernel(user_indices, item_indices, user_table, item_table):
    return _rec_sc(user_indices.astype(jnp.int32),
                   item_indices.astype(jnp.int32),
                   user_table, item_table)


# speedup vs baseline: 2.3470x; 2.3470x over previous
import functools

import jax
import jax.numpy as jnp
from jax import lax
from jax.experimental import pallas as pl
from jax.experimental.pallas import tpu as pltpu
from jax.experimental.pallas import tpu_sc as plsc

BATCH = 16384
D = 64
L = 16
NC = 2
NS = 16
NW = NC * NS
CHUNK = BATCH // NW
GUSERS = 8
NIDX = 128

_mesh = plsc.VectorSubcoreMesh(core_axis_name="c", subcore_axis_name="s")


@functools.partial(
    pl.kernel,
    mesh=_mesh,
    compiler_params=pltpu.CompilerParams(needs_layout_passes=False),
    out_type=jax.ShapeDtypeStruct((BATCH,), jnp.float32),
    scratch_types=[
        pltpu.VMEM((CHUNK // NIDX, NIDX), jnp.int32),
        pltpu.VMEM((CHUNK // NIDX, NIDX), jnp.int32),
        pltpu.VMEM((4, D, 128), jnp.float32),
        pltpu.VMEM((4, D, 128), jnp.float32),
        pltpu.VMEM((CHUNK,), jnp.float32),
        pltpu.SemaphoreType.DMA,
    ],
)
def _rec_sc(uidx_hbm, iidx_hbm, utabT_hbm, itabT_hbm, out_hbm,
            uidx_v, iidx_v, ubuf_v, ibuf_v, res_v, sem):
    wid = lax.axis_index("s") * NC + lax.axis_index("c")
    base = wid * CHUNK

    for j in range(CHUNK // NIDX):
        pltpu.sync_copy(uidx_hbm.at[pl.ds(base + j * NIDX, NIDX)], uidx_v.at[j])
        pltpu.sync_copy(iidx_hbm.at[pl.ds(base + j * NIDX, NIDX)], iidx_v.at[j])

    iota16 = lax.iota(jnp.int32, L)
    c16 = [jnp.full((L,), 0, jnp.int32) + iota16 + 16 * k for k in range(D // L)]
    zero = jnp.zeros((L,), jnp.float32)

    def grp_body(g, _):
        ridx_u = uidx_v[g // (NIDX // L), pl.ds((g % (NIDX // L)) * L, L)]
        ridx_i = iidx_v[g // (NIDX // L), pl.ds((g % (NIDX // L)) * L, L)]
        out16 = zero
        for t in range(4):
            copies = []
            for p in range(4):
                r_u = ridx_u[t * 4 + p]
                r_i = ridx_i[t * 4 + p]
                copies.append(pltpu.async_copy(
                    utabT_hbm.at[pl.ds(0, D),
                                 pl.ds(pl.multiple_of((r_u // 128) * 128, 128), 128)],
                    ubuf_v.at[p], sem))
                copies.append(pltpu.async_copy(
                    itabT_hbm.at[pl.ds(0, D),
                                 pl.ds(pl.multiple_of((r_i // 128) * 128, 128), 128)],
                    ibuf_v.at[p], sem))
            for c in copies:
                c.wait()

            for p in range(4):
                o_u = jnp.full((L,), 0, jnp.int32) + (ridx_u[t * 4 + p] % 128)
                o_i = jnp.full((L,), 0, jnp.int32) + (ridx_i[t * 4 + p] % 128)
                acc = zero
                for k in range(D // L):
                    uvec = plsc.load_gather(ubuf_v.at[p], [c16[k], o_u])
                    ivec = plsc.load_gather(ibuf_v.at[p], [c16[k], o_i])
                    acc = acc + jnp.maximum(uvec, 0.0) * jnp.maximum(ivec, 0.0)
                out16 = jnp.where(iota16 == (t * 4 + p), jnp.sum(acc), out16)
        res_v[pl.ds(g * L, L)] = 4.0 / (1.0 + jnp.exp(-out16)) + 1.0
        return 0

    lax.fori_loop(0, CHUNK // L, grp_body, 0)

    pltpu.sync_copy(res_v, out_hbm.at[pl.ds(base, CHUNK)])


def kernel(user_indices, item_indices, user_table, item_table):
    return _rec_sc(user_indices.astype(jnp.int32),
                   item_indices.astype(jnp.int32),
                   user_table.T, item_table.T)
